# baseline (speedup 1.0000x reference)
import numpy as np
import jax
import jax.numpy as jnp
from jax import lax
from jax.experimental import pallas as pl
from jax.experimental.pallas import tpu as pltpu
from jax.experimental.pallas import tpu_sc as plsc

N = 10000
NPAD = 10240
E = 160000
D = 128
H = 10
DH = 13
DHP = 16
FW = H * DHP
HP = 16

RB = 512
EB = 2000
_GRID_N = NPAD // RB

EPT = E // 32
EPC = E // 2
CH = 128
NCH = EPT // CH
TL = EPT - NCH * CH
RPT = NPAD // 16
_GRID_E = E // EB

_f32 = jnp.float32


def _elu(x):
    return jnp.where(x > 0, x, jnp.exp(jnp.minimum(x, 0.0)) - 1.0)


def _lnorm(x, g, b):
    m = jnp.mean(x, axis=-1, keepdims=True)
    v = jnp.mean(jnp.square(x - m), axis=-1, keepdims=True)
    return (x - m) / jnp.sqrt(v + 1e-5) * g + b


W1 = 8 * DHP
W2 = 2 * DHP

_G1_NP = np.zeros((W1, HP), np.float32)
_G2_NP = np.zeros((W2, HP), np.float32)
for _h in range(8):
    _G1_NP[_h * DHP:_h * DHP + DH, _h] = 1.0
for _h in range(8, H):
    _G2_NP[(_h - 8) * DHP:(_h - 8) * DHP + DH, _h] = 1.0
_GE1_NP = _G1_NP / np.sqrt(float(DH))
_GE2_NP = _G2_NP / np.sqrt(float(DH))
_GX1_NP = _G1_NP.T.copy()
_GX2_NP = _G2_NP.T.copy()


_SC_CACHE = {}


def _sc_mesh():
    if "mesh" not in _SC_CACHE:
        _SC_CACHE["mesh"] = plsc.VectorSubcoreMesh(
            core_axis_name="c", subcore_axis_name="s")
    return _SC_CACHE["mesh"]


def _sc_kernel(name, body, out_type, scratch_types, tc_tiling=False):
    if name not in _SC_CACHE:
        _SC_CACHE[name] = pl.kernel(
            body, out_type=out_type, mesh=_sc_mesh(),
            scratch_types=scratch_types,
            compiler_params=pltpu.CompilerParams(
                use_tc_tiling_on_sc=tc_tiling))
    return _SC_CACHE[name]


def _deg_body(dst_hbm, out_hbm, dsti, dsti1, dsti8, ones_v, zb, acc, s0, s1):
    c = lax.axis_index("c")
    s = lax.axis_index("s")

    def fill(i, _):
        ones_v[i, :] = jnp.ones((16,), _f32)
        zb[i, :] = jnp.zeros((16,), _f32)
        return 0
    lax.fori_loop(0, CH, fill, 0)
    for r in range(RPT // CH):
        pltpu.sync_copy(zb, acc.at[pl.ds(s * RPT + r * CH, CH)])
    plsc.subcore_barrier()

    eb = c * EPC + s * EPT

    def ldd(j, dref, sm):
        pltpu.async_copy(dst_hbm.at[pl.ds(eb + j * CH, CH)], dref, sm)

    def wtd(j, dref, sm):
        pltpu.make_async_copy(dst_hbm.at[pl.ds(eb + j * CH, CH)], dref,
                              sm).wait()

    ldd(0, dsti, s0)

    def body(k, _):
        j = 2 * k
        ldd(j + 1, dsti1, s1)
        wtd(j, dsti, s0)
        pltpu.sync_copy(ones_v, acc.at[dsti], add=True)
        ldd(j + 2, dsti, s0)
        wtd(j + 1, dsti1, s1)
        pltpu.sync_copy(ones_v, acc.at[dsti1], add=True)
        return 0
    lax.fori_loop(0, (NCH - 1) // 2, body, 0)
    wtd(NCH - 1, dsti, s0)
    pltpu.sync_copy(ones_v, acc.at[dsti], add=True)

    pltpu.sync_copy(dst_hbm.at[pl.ds(eb + NCH * CH, TL)], dsti8)
    pltpu.sync_copy(ones_v.at[pl.ds(0, TL)], acc.at[dsti8], add=True)

    plsc.subcore_barrier()
    pltpu.sync_copy(acc.at[pl.ds(s * RPT, RPT)],
                    out_hbm.at[pl.ds(c * NPAD + s * RPT, RPT)])


def _deg_call():
    return _sc_kernel(
        "deg",
        _deg_body,
        out_type=jax.ShapeDtypeStruct((2 * NPAD, HP), _f32),
        scratch_types=[
            pltpu.VMEM((CH,), jnp.int32),
            pltpu.VMEM((CH,), jnp.int32),
            pltpu.VMEM((TL,), jnp.int32),
            pltpu.VMEM((CH, HP), _f32),
            pltpu.VMEM((CH, HP), _f32),
            pltpu.VMEM_SHARED((NPAD, HP), _f32),
            pltpu.SemaphoreType.DMA,
            pltpu.SemaphoreType.DMA,
        ],
    )


def _segsum_body(hn_hbm, src_hbm, dst_hbm, out_hbm,
                 srci, dsti, srci1, dsti1, srci8, dsti8,
                 rows, rows1, rows8, zb, acc, sem, sem1):
    c = lax.axis_index("c")
    s = lax.axis_index("s")

    def zf(i, _):
        for kk in range(D // 16):
            zb[i, pl.ds(kk * 16, 16)] = jnp.zeros((16,), _f32)
        return 0
    lax.fori_loop(0, 64, zf, 0)
    for r in range(RPT // 64):
        pltpu.sync_copy(zb, acc.at[pl.ds(s * RPT + r * 64, 64)])
    plsc.subcore_barrier()

    eb = c * EPC + s * EPT

    def ld(j, sref, dref):
        pltpu.sync_copy(src_hbm.at[pl.ds(eb + j * CH, CH)], sref)
        pltpu.sync_copy(dst_hbm.at[pl.ds(eb + j * CH, CH)], dref)

    ld(0, srci, dsti)
    pltpu.async_copy(hn_hbm.at[srci], rows, sem)

    def body(k, _):
        j = 2 * k
        ld(j + 1, srci1, dsti1)
        pltpu.async_copy(hn_hbm.at[srci1], rows1, sem1)
        pltpu.make_async_copy(hn_hbm.at[srci], rows, sem).wait()
        pltpu.sync_copy(rows, acc.at[dsti], add=True)
        ld(j + 2, srci, dsti)
        pltpu.async_copy(hn_hbm.at[srci], rows, sem)
        pltpu.make_async_copy(hn_hbm.at[srci1], rows1, sem1).wait()
        pltpu.sync_copy(rows1, acc.at[dsti1], add=True)
        return 0
    lax.fori_loop(0, (NCH - 1) // 2, body, 0)
    pltpu.make_async_copy(hn_hbm.at[srci], rows, sem).wait()
    pltpu.sync_copy(rows, acc.at[dsti], add=True)

    pltpu.sync_copy(src_hbm.at[pl.ds(eb + NCH * CH, TL)], srci8)
    pltpu.sync_copy(dst_hbm.at[pl.ds(eb + NCH * CH, TL)], dsti8)
    pltpu.async_copy(hn_hbm.at[srci8], rows8, sem).wait()
    pltpu.sync_copy(rows8, acc.at[dsti8], add=True)

    plsc.subcore_barrier()
    pltpu.sync_copy(acc.at[pl.ds(s * RPT, RPT)],
                    out_hbm.at[pl.ds(c * NPAD + s * RPT, RPT)])


def _segsum_call():
    return _sc_kernel(
        "segsum",
        _segsum_body,
        out_type=jax.ShapeDtypeStruct((2 * NPAD, D), _f32),
        scratch_types=[
            pltpu.VMEM((CH,), jnp.int32),
            pltpu.VMEM((CH,), jnp.int32),
            pltpu.VMEM((CH,), jnp.int32),
            pltpu.VMEM((CH,), jnp.int32),
            pltpu.VMEM((TL,), jnp.int32),
            pltpu.VMEM((TL,), jnp.int32),
            pltpu.VMEM((CH, D), _f32),
            pltpu.VMEM((CH, D), _f32),
            pltpu.VMEM((TL, D), _f32),
            pltpu.VMEM((64, D), _f32),
            pltpu.VMEM_SHARED((NPAD, D), _f32),
            pltpu.SemaphoreType.DMA,
            pltpu.SemaphoreType.DMA,
        ],
    )


def _gath_body(f1_hbm, f2_hbm, src_hbm, dst_hbm,
               fs1_hbm, fs2_hbm, fd1_hbm, fd2_hbm,
               srci, dsti, srci1, dsti1, srci8, dsti8,
               s1b, s2b, d1b, d2b, s1b1, s2b1, d1b1, d2b1,
               s1b8, s2b8, d1b8, d2b8, sem, sem2, sem3, sem4):
    c = lax.axis_index("c")
    s = lax.axis_index("s")
    eb = (c * 16 + s) * EPT

    def ld(j, sref, dref):
        pltpu.sync_copy(src_hbm.at[pl.ds(eb + j * CH, CH)], sref)
        pltpu.sync_copy(dst_hbm.at[pl.ds(eb + j * CH, CH)], dref)

    def gat(sref, dref, b1, b2, b3, b4, sm, sm2):
        pltpu.async_copy(f1_hbm.at[sref], b1, sm)
        pltpu.async_copy(f2_hbm.at[sref], b2, sm)
        pltpu.async_copy(f1_hbm.at[dref], b3, sm2)
        pltpu.async_copy(f2_hbm.at[dref], b4, sm2)

    def wr(j, sref, dref, b1, b2, b3, b4, sm, sm2):
        pltpu.make_async_copy(f1_hbm.at[sref], b1, sm).wait()
        pltpu.make_async_copy(f2_hbm.at[sref], b2, sm).wait()
        pltpu.make_async_copy(f1_hbm.at[dref], b3, sm2).wait()
        pltpu.make_async_copy(f2_hbm.at[dref], b4, sm2).wait()
        pltpu.sync_copy(b1, fs1_hbm.at[pl.ds(eb + j * CH, CH)])
        pltpu.sync_copy(b2, fs2_hbm.at[pl.ds(eb + j * CH, CH)])
        pltpu.sync_copy(b3, fd1_hbm.at[pl.ds(eb + j * CH, CH)])
        pltpu.sync_copy(b4, fd2_hbm.at[pl.ds(eb + j * CH, CH)])

    ld(0, srci, dsti)
    gat(srci, dsti, s1b, s2b, d1b, d2b, sem, sem2)

    def body(k, _):
        j = 2 * k
        ld(j + 1, srci1, dsti1)
        gat(srci1, dsti1, s1b1, s2b1, d1b1, d2b1, sem3, sem4)
        wr(j, srci, dsti, s1b, s2b, d1b, d2b, sem, sem2)
        ld(j + 2, srci, dsti)
        gat(srci, dsti, s1b, s2b, d1b, d2b, sem, sem2)
        wr(j + 1, srci1, dsti1, s1b1, s2b1, d1b1, d2b1, sem3, sem4)
        return 0
    lax.fori_loop(0, (NCH - 1) // 2, body, 0)
    wr(NCH - 1, srci, dsti, s1b, s2b, d1b, d2b, sem, sem2)

    pltpu.sync_copy(src_hbm.at[pl.ds(eb + NCH * CH, TL)], srci8)
    pltpu.sync_copy(dst_hbm.at[pl.ds(eb + NCH * CH, TL)], dsti8)
    gat(srci8, dsti8, s1b8, s2b8, d1b8, d2b8, sem, sem2)
    pltpu.make_async_copy(f1_hbm.at[srci8], s1b8, sem).wait()
    pltpu.make_async_copy(f2_hbm.at[srci8], s2b8, sem).wait()
    pltpu.make_async_copy(f1_hbm.at[dsti8], d1b8, sem2).wait()
    pltpu.make_async_copy(f2_hbm.at[dsti8], d2b8, sem2).wait()
    pltpu.sync_copy(s1b8, fs1_hbm.at[pl.ds(eb + NCH * CH, TL)])
    pltpu.sync_copy(s2b8, fs2_hbm.at[pl.ds(eb + NCH * CH, TL)])
    pltpu.sync_copy(d1b8, fd1_hbm.at[pl.ds(eb + NCH * CH, TL)])
    pltpu.sync_copy(d2b8, fd2_hbm.at[pl.ds(eb + NCH * CH, TL)])


def _gath_call():
    return _sc_kernel(
        "gath",
        _gath_body,
        out_type=(jax.ShapeDtypeStruct((E, W1), jnp.bfloat16),
                  jax.ShapeDtypeStruct((E, W2), _f32),
                  jax.ShapeDtypeStruct((E, W1), jnp.bfloat16),
                  jax.ShapeDtypeStruct((E, W2), _f32)),
        scratch_types=[
            pltpu.VMEM((CH,), jnp.int32),
            pltpu.VMEM((CH,), jnp.int32),
            pltpu.VMEM((CH,), jnp.int32),
            pltpu.VMEM((CH,), jnp.int32),
            pltpu.VMEM((TL,), jnp.int32),
            pltpu.VMEM((TL,), jnp.int32),
            pltpu.VMEM((CH, W1), jnp.bfloat16),
            pltpu.VMEM((CH, W2), _f32),
            pltpu.VMEM((CH, W1), jnp.bfloat16),
            pltpu.VMEM((CH, W2), _f32),
            pltpu.VMEM((CH, W1), jnp.bfloat16),
            pltpu.VMEM((CH, W2), _f32),
            pltpu.VMEM((CH, W1), jnp.bfloat16),
            pltpu.VMEM((CH, W2), _f32),
            pltpu.VMEM((TL, W1), jnp.bfloat16),
            pltpu.VMEM((TL, W2), _f32),
            pltpu.VMEM((TL, W1), jnp.bfloat16),
            pltpu.VMEM((TL, W2), _f32),
            pltpu.SemaphoreType.DMA,
            pltpu.SemaphoreType.DMA,
            pltpu.SemaphoreType.DMA,
            pltpu.SemaphoreType.DMA,
        ],
    )


CHS = 40
NCHS = EPT // CHS


def _attscat_body(m1_hbm, m2_hbm, a_hbm, dst_hbm, z_hbm,
                  rs1_hbm, rs2_hbm, dn_hbm,
                  dsti, dsti1, m1b, m1b1, m2b, m2b1, ab, ab1, zb2, zba,
                  accm1, accm2, acca, sd0, sd1):
    c = lax.axis_index("c")
    s = lax.axis_index("s")

    pltpu.sync_copy(z_hbm.at[pl.ds(s * RPT, RPT)],
                    accm1.at[pl.ds(s * RPT, RPT)])

    def zf(i, _):
        zb2[i, pl.ds(0, 16)] = jnp.zeros((16,), _f32)
        zb2[i, pl.ds(16, 16)] = jnp.zeros((16,), _f32)
        zba[i, :] = jnp.zeros((16,), _f32)
        return 0
    lax.fori_loop(0, CHS, zf, 0)

    def zcp(r, _):
        pltpu.sync_copy(zb2, accm2.at[pl.ds(s * RPT + r * CHS, CHS)])
        pltpu.sync_copy(zba, acca.at[pl.ds(s * RPT + r * CHS, CHS)])
        return 0
    lax.fori_loop(0, RPT // CHS, zcp, 0)
    plsc.subcore_barrier()

    eb = c * EPC + s * EPT

    def lda(j, dref, b1, b2, ba, sm):
        rb = eb + j * CHS
        pltpu.async_copy(dst_hbm.at[pl.ds(rb, CHS)], dref, sm)
        pltpu.async_copy(m1_hbm.at[pl.ds(rb, CHS)], b1, sm)
        pltpu.async_copy(m2_hbm.at[pl.ds(rb, CHS)], b2, sm)
        pltpu.async_copy(a_hbm.at[pl.ds(rb, CHS)], ba, sm)

    def wsc(j, dref, b1, b2, ba, sm):
        rb = eb + j * CHS
        pltpu.make_async_copy(dst_hbm.at[pl.ds(rb, CHS)], dref, sm).wait()
        pltpu.make_async_copy(m1_hbm.at[pl.ds(rb, CHS)], b1, sm).wait()
        pltpu.make_async_copy(m2_hbm.at[pl.ds(rb, CHS)], b2, sm).wait()
        pltpu.make_async_copy(a_hbm.at[pl.ds(rb, CHS)], ba, sm).wait()
        pltpu.sync_copy(b1, accm1.at[dref], add=True)
        pltpu.sync_copy(b2, accm2.at[dref], add=True)
        pltpu.sync_copy(ba, acca.at[dref], add=True)

    lda(0, dsti, m1b, m2b, ab, sd0)

    def body(k, _):
        j = 2 * k
        lda(j + 1, dsti1, m1b1, m2b1, ab1, sd1)
        wsc(j, dsti, m1b, m2b, ab, sd0)
        lda(j + 2, dsti, m1b, m2b, ab, sd0)
        wsc(j + 1, dsti1, m1b1, m2b1, ab1, sd1)
        return 0
    lax.fori_loop(0, (NCHS - 1) // 2, body, 0)
    wsc(NCHS - 1, dsti, m1b, m2b, ab, sd0)

    plsc.subcore_barrier()
    pltpu.sync_copy(accm1.at[pl.ds(s * RPT, RPT)],
                    rs1_hbm.at[pl.ds(c * NPAD + s * RPT, RPT)])
    pltpu.sync_copy(accm2.at[pl.ds(s * RPT, RPT)],
                    rs2_hbm.at[pl.ds(c * NPAD + s * RPT, RPT)])
    pltpu.sync_copy(acca.at[pl.ds(s * RPT, RPT)],
                    dn_hbm.at[pl.ds(c * NPAD + s * RPT, RPT)])


def _attscat_call():
    return _sc_kernel(
        "attscat",
        _attscat_body,
        out_type=(jax.ShapeDtypeStruct((2 * NPAD, W1), _f32),
                  jax.ShapeDtypeStruct((2 * NPAD, W2), _f32),
                  jax.ShapeDtypeStruct((2 * NPAD, HP), _f32)),
        scratch_types=[
            pltpu.VMEM((CHS,), jnp.int32),
            pltpu.VMEM((CHS,), jnp.int32),
            pltpu.VMEM((CHS, W1), _f32),
            pltpu.VMEM((CHS, W1), _f32),
            pltpu.VMEM((CHS, W2), _f32),
            pltpu.VMEM((CHS, W2), _f32),
            pltpu.VMEM((CHS, HP), _f32),
            pltpu.VMEM((CHS, HP), _f32),
            pltpu.VMEM((CHS, W2), _f32),
            pltpu.VMEM((CHS, HP), _f32),
            pltpu.VMEM_SHARED((NPAD, W1), _f32),
            pltpu.VMEM_SHARED((NPAD, W2), _f32),
            pltpu.VMEM_SHARED((NPAD, HP), _f32),
            pltpu.SemaphoreType.DMA,
            pltpu.SemaphoreType.DMA,
        ],
    )


def _row_spec(w):
    return pl.BlockSpec((RB, w), lambda i: (i, 0))


def _full_spec(shape):
    nd = len(shape)
    return pl.BlockSpec(shape, lambda i: (0,) * nd)


def _dot(a, b):
    return jnp.dot(a, b, preferred_element_type=_f32)


def _in_body(x_ref, wi, bi, g0, b0, hn_ref):
    h = _dot(x_ref[...], wi[...]) + bi[...]
    hn_ref[...] = _lnorm(h, g0[...], b0[...])


_in_call = pl.pallas_call(
    _in_body,
    grid=(_GRID_N,),
    in_specs=[_row_spec(D), _full_spec((D, D)), _full_spec((1, D)),
              _full_spec((1, D)), _full_spec((1, D))],
    out_specs=_row_spec(D),
    out_shape=jax.ShapeDtypeStruct((NPAD, D), _f32),
)


def _make_conv_fin(has_feat):
    def body(hn_ref, sp_ref, degp_ref, wself, wneigh, bias, ig, ib, siw, sib,
             ng, nb, *rest):
        if has_feat:
            watt1, watt2 = rest[0], rest[1]
            outs = rest[2:]
        else:
            outs = rest
        hn = hn_ref[...]
        spv = sp_ref[...]
        sv = spv[0] + spv[1]
        dgv = degp_ref[...]
        deg = dgv[0, :, 0:1] + dgv[1, :, 0:1]
        mean = sv / jnp.maximum(deg, 1.0)
        conv = _dot(hn, wself[...]) + _dot(mean, wneigh[...]) + bias[...]
        h2 = _lnorm(conv + hn, ig[...], ib[...])
        hnew = h2 + _elu(_dot(h2, siw[...]) + sib[...])
        hn2 = _lnorm(hnew, ng[...], nb[...])
        outs[0][...] = hn2
        if has_feat:
            outs[1][...] = _dot(hn2, watt1[...]).astype(jnp.bfloat16)
            outs[2][...] = _dot(hn2, watt2[...])
    return body


def _conv_fin_call(has_feat):
    in_specs = [
        _row_spec(D),
        pl.BlockSpec((2, RB, D), lambda i: (0, i, 0)),
        pl.BlockSpec((2, RB, HP), lambda i: (0, i, 0)),
        _full_spec((D, D)), _full_spec((D, D)), _full_spec((1, D)),
        _full_spec((1, D)), _full_spec((1, D)),
        _full_spec((D, D)), _full_spec((1, D)),
        _full_spec((1, D)), _full_spec((1, D)),
    ]
    out_specs = [_row_spec(D)]
    out_shape = [jax.ShapeDtypeStruct((NPAD, D), _f32)]
    if has_feat:
        in_specs.append(_full_spec((D, W1)))
        in_specs.append(_full_spec((D, W2)))
        out_specs.append(_row_spec(W1))
        out_shape.append(jax.ShapeDtypeStruct((NPAD, W1), jnp.bfloat16))
        out_specs.append(_row_spec(W2))
        out_shape.append(jax.ShapeDtypeStruct((NPAD, W2), _f32))
    return pl.pallas_call(
        _make_conv_fin(has_feat),
        grid=(_GRID_N,),
        in_specs=in_specs,
        out_specs=out_specs,
        out_shape=out_shape,
    )


def _edge_body(fs1_ref, fs2_ref, fd1_ref, fd2_ref, ge1, ge2, gx1, gx2,
               m1_ref, m2_ref, a_ref):
    fs1 = fs1_ref[...].astype(_f32)
    fs2 = fs2_ref[...]
    prod1 = fs1 * fd1_ref[...].astype(_f32)
    prod2 = fs2 * fd2_ref[...]
    a = jnp.exp(_dot(prod1, ge1[...]) + _dot(prod2, ge2[...]))
    a_ref[...] = a
    m1_ref[...] = _dot(a, gx1[...]) * fs1
    m2_ref[...] = _dot(a, gx2[...]) * fs2


_edge_call = pl.pallas_call(
    _edge_body,
    grid=(_GRID_E,),
    in_specs=[pl.BlockSpec((EB, W1), lambda i: (i, 0)),
              pl.BlockSpec((EB, W2), lambda i: (i, 0)),
              pl.BlockSpec((EB, W1), lambda i: (i, 0)),
              pl.BlockSpec((EB, W2), lambda i: (i, 0)),
              _full_spec((W1, HP)), _full_spec((W2, HP)),
              _full_spec((HP, W1)), _full_spec((HP, W2))],
    out_specs=[pl.BlockSpec((EB, W1), lambda i: (i, 0)),
               pl.BlockSpec((EB, W2), lambda i: (i, 0)),
               pl.BlockSpec((EB, HP), lambda i: (i, 0))],
    out_shape=[jax.ShapeDtypeStruct((E, W1), _f32),
               jax.ShapeDtypeStruct((E, W2), _f32),
               jax.ShapeDtypeStruct((E, HP), _f32)],
)


def _make_att_fin(is_final):
    def body(hn_ref, rs1_ref, rs2_ref, dn_ref, gx1, gx2, hrw1, hrw2, hrb,
             ig, ib, f1w, f1b, f2w, f2b, *rest):
        hn = hn_ref[...]
        rs1v = rs1_ref[...]
        rs1 = rs1v[0] + rs1v[1]
        rs2v = rs2_ref[...]
        rs2 = rs2v[0] + rs2v[1]
        dnv = dn_ref[...]
        dn = dnv[0] + dnv[1]
        rst1 = rs1 / jnp.maximum(_dot(dn, gx1[...]), 1e-30)
        rst2 = rs2 / jnp.maximum(_dot(dn, gx2[...]), 1e-30)
        ho = (_dot(_elu(rst1), hrw1[...]) + _dot(_elu(rst2), hrw2[...])
              + hrb[...])
        h2 = _lnorm(ho + hn, ig[...], ib[...])
        ff = _elu(_dot(_elu(_dot(h2, f1w[...]) + f1b[...]), f2w[...])
                  + f2b[...])
        hnew = h2 + ff
        if is_final:
            wout, bout, out_ref = rest
            out_ref[...] = _dot(hnew, wout[...]) + bout[...]
        else:
            ng, nb, watt1, watt2, hn_out, f1_out, f2_out = rest
            hn2 = _lnorm(hnew, ng[...], nb[...])
            hn_out[...] = hn2
            f1_out[...] = _dot(hn2, watt1[...]).astype(jnp.bfloat16)
            f2_out[...] = _dot(hn2, watt2[...])
    return body


def _att_fin_call(is_final):
    in_specs = [
        _row_spec(D),
        pl.BlockSpec((2, RB, W1), lambda i: (0, i, 0)),
        pl.BlockSpec((2, RB, W2), lambda i: (0, i, 0)),
        pl.BlockSpec((2, RB, HP), lambda i: (0, i, 0)),
        _full_spec((HP, W1)), _full_spec((HP, W2)),
        _full_spec((W1, D)), _full_spec((W2, D)), _full_spec((1, D)),
        _full_spec((1, D)), _full_spec((1, D)),
        _full_spec((D, 4 * D)), _full_spec((1, 4 * D)),
        _full_spec((4 * D, D)), _full_spec((1, D)),
    ]
    if is_final:
        in_specs += [_full_spec((D, D)), _full_spec((1, D))]
        out_specs = _row_spec(D)
        out_shape = jax.ShapeDtypeStruct((NPAD, D), _f32)
    else:
        in_specs += [_full_spec((1, D)), _full_spec((1, D)),
                     _full_spec((D, W1)), _full_spec((D, W2))]
        out_specs = [_row_spec(D), _row_spec(W1), _row_spec(W2)]
        out_shape = [jax.ShapeDtypeStruct((NPAD, D), _f32),
                     jax.ShapeDtypeStruct((NPAD, W1), jnp.bfloat16),
                     jax.ShapeDtypeStruct((NPAD, W2), _f32)]
    return pl.pallas_call(
        _make_att_fin(is_final),
        grid=(_GRID_N,),
        in_specs=in_specs,
        out_specs=out_specs,
        out_shape=out_shape,
    )


_convfin_plain = _conv_fin_call(False)
_convfin_feat = _conv_fin_call(True)
_attfin_mid = _att_fin_call(False)
_attfin_last = _att_fin_call(True)


def _watt_pad(p):
    w = p["W_att"].reshape(D, H, DH)
    wp = jnp.pad(w, ((0, 0), (0, 0), (0, DHP - DH)))
    return (wp[:, :8].reshape(D, W1), wp[:, 8:].reshape(D, W2))


def _hrw_pad(p):
    w = p["hr_W"].reshape(H, DH, D)
    wp = jnp.pad(w, ((0, 0), (0, DHP - DH), (0, 0)))
    return (wp[:8].reshape(W1, D), wp[8:].reshape(W2, D))


def kernel(x, params, edge_index):
    src = edge_index[0]
    dst = edge_index[1]
    xp = jnp.pad(x, ((0, NPAD - N), (0, 0)))
    r1 = lambda v: v.reshape(1, -1)
    ge1 = jnp.asarray(_GE1_NP)
    ge2 = jnp.asarray(_GE2_NP)
    gx1 = jnp.asarray(_GX1_NP)
    gx2 = jnp.asarray(_GX2_NP)
    z1 = jnp.zeros((NPAD, W1), _f32)

    degp = _deg_call()(dst).reshape(2, NPAD, HP)

    p0 = params["conv0"]
    hn = _in_call(xp, params["W_in"], r1(params["b_in"]),
                  r1(p0["ln_g"]), r1(p0["ln_b"]))

    feat = None
    for i in range(3):
        p = params["conv%d" % i]
        sp = _segsum_call()(hn, src, dst).reshape(2, NPAD, D)
        common = (hn, sp, degp, p["Wself"], p["Wneigh"], r1(p["bias"]),
                  r1(p["iln_g"]), r1(p["iln_b"]), p["si_W"], r1(p["si_b"]))
        if i < 2:
            q = params["conv%d" % (i + 1)]
            (hn,) = _convfin_plain(*common, r1(q["ln_g"]), r1(q["ln_b"]))
        else:
            q = params["att0"]
            wa1, wa2 = _watt_pad(q)
            hn, feat1, feat2 = _convfin_feat(*common, r1(q["ln_g"]),
                                             r1(q["ln_b"]), wa1, wa2)

    for j in range(3):
        p = params["att%d" % j]
        fs1, fs2, fd1, fd2 = _gath_call()(feat1, feat2, src, dst)
        m1, m2, a = _edge_call(fs1, fs2, fd1, fd2, ge1, ge2, gx1, gx2)
        rs1, rs2, dnp = _attscat_call()(m1, m2, a, dst, z1)
        hr1, hr2 = _hrw_pad(p)
        common = (hn, rs1.reshape(2, NPAD, W1), rs2.reshape(2, NPAD, W2),
                  dnp.reshape(2, NPAD, HP),
                  gx1, gx2, hr1, hr2, r1(p["hr_b"]),
                  r1(p["iln_g"]), r1(p["iln_b"]),
                  p["ff1_W"], r1(p["ff1_b"]), p["ff2_W"], r1(p["ff2_b"]))
        if j < 2:
            q = params["att%d" % (j + 1)]
            wa1, wa2 = _watt_pad(q)
            hn, feat1, feat2 = _attfin_mid(*common, r1(q["ln_g"]),
                                           r1(q["ln_b"]), wa1, wa2)
        else:
            out = _attfin_last(*common, params["W_out"], r1(params["b_out"]))

    return out[:N]


# speedup vs baseline: 1.3456x; 1.3456x over previous
import numpy as np
import jax
import jax.numpy as jnp
from jax import lax
from jax.experimental import pallas as pl
from jax.experimental.pallas import tpu as pltpu
from jax.experimental.pallas import tpu_sc as plsc

N = 10000
NPAD = 10240
E = 160000
D = 128
H = 10
DH = 13
DHP = 16
FW = H * DHP
HP = 16

RB = 512
EB = 2000
_GRID_N = NPAD // RB

EPT = E // 32
EPC = E // 2
CH = 128
NCH = EPT // CH
TL = EPT - NCH * CH
RPT = NPAD // 16
_GRID_E = E // EB

_f32 = jnp.float32


def _elu(x):
    return jnp.where(x > 0, x, jnp.exp(jnp.minimum(x, 0.0)) - 1.0)


def _lnorm(x, g, b):
    m = jnp.mean(x, axis=-1, keepdims=True)
    v = jnp.mean(jnp.square(x - m), axis=-1, keepdims=True)
    return (x - m) / jnp.sqrt(v + 1e-5) * g + b


W1 = 8 * DHP
W2 = 2 * DHP

_G1_NP = np.zeros((W1, HP), np.float32)
_G2_NP = np.zeros((W2, HP), np.float32)
for _h in range(8):
    _G1_NP[_h * DHP:_h * DHP + DH, _h] = 1.0
for _h in range(8, H):
    _G2_NP[(_h - 8) * DHP:(_h - 8) * DHP + DH, _h] = 1.0
_GE1_NP = _G1_NP / np.sqrt(float(DH))
_GE2_NP = _G2_NP / np.sqrt(float(DH))
_GX1_NP = _G1_NP.T.copy()
_GX2_NP = _G2_NP.T.copy()


_SC_CACHE = {}


def _sc_mesh():
    if "mesh" not in _SC_CACHE:
        _SC_CACHE["mesh"] = plsc.VectorSubcoreMesh(
            core_axis_name="c", subcore_axis_name="s")
    return _SC_CACHE["mesh"]


def _sc_kernel(name, body, out_type, scratch_types, tc_tiling=False):
    if name not in _SC_CACHE:
        _SC_CACHE[name] = pl.kernel(
            body, out_type=out_type, mesh=_sc_mesh(),
            scratch_types=scratch_types,
            compiler_params=pltpu.CompilerParams(
                use_tc_tiling_on_sc=tc_tiling))
    return _SC_CACHE[name]


def _deg_body(dst_hbm, out_hbm, dsti, dsti1, dsti8, ones_v, zb, acc, s0, s1):
    c = lax.axis_index("c")
    s = lax.axis_index("s")

    def fill(i, _):
        ones_v[i, :] = jnp.ones((16,), _f32)
        zb[i, :] = jnp.zeros((16,), _f32)
        return 0
    lax.fori_loop(0, CH, fill, 0)
    for r in range(RPT // CH):
        pltpu.sync_copy(zb, acc.at[pl.ds(s * RPT + r * CH, CH)])
    plsc.subcore_barrier()

    eb = c * EPC + s * EPT

    def ldd(j, dref, sm):
        pltpu.async_copy(dst_hbm.at[pl.ds(eb + j * CH, CH)], dref, sm)

    def wtd(j, dref, sm):
        pltpu.make_async_copy(dst_hbm.at[pl.ds(eb + j * CH, CH)], dref,
                              sm).wait()

    ldd(0, dsti, s0)

    def body(k, _):
        j = 2 * k
        ldd(j + 1, dsti1, s1)
        wtd(j, dsti, s0)
        pltpu.sync_copy(ones_v, acc.at[dsti], add=True)
        ldd(j + 2, dsti, s0)
        wtd(j + 1, dsti1, s1)
        pltpu.sync_copy(ones_v, acc.at[dsti1], add=True)
        return 0
    lax.fori_loop(0, (NCH - 1) // 2, body, 0)
    wtd(NCH - 1, dsti, s0)
    pltpu.sync_copy(ones_v, acc.at[dsti], add=True)

    pltpu.sync_copy(dst_hbm.at[pl.ds(eb + NCH * CH, TL)], dsti8)
    pltpu.sync_copy(ones_v.at[pl.ds(0, TL)], acc.at[dsti8], add=True)

    plsc.subcore_barrier()
    pltpu.sync_copy(acc.at[pl.ds(s * RPT, RPT)],
                    out_hbm.at[pl.ds(c * NPAD + s * RPT, RPT)])


def _deg_call():
    return _sc_kernel(
        "deg",
        _deg_body,
        out_type=jax.ShapeDtypeStruct((2 * NPAD, HP), _f32),
        scratch_types=[
            pltpu.VMEM((CH,), jnp.int32),
            pltpu.VMEM((CH,), jnp.int32),
            pltpu.VMEM((TL,), jnp.int32),
            pltpu.VMEM((CH, HP), _f32),
            pltpu.VMEM((CH, HP), _f32),
            pltpu.VMEM_SHARED((NPAD, HP), _f32),
            pltpu.SemaphoreType.DMA,
            pltpu.SemaphoreType.DMA,
        ],
    )


def _segsum_body(hn_hbm, src_hbm, dst_hbm, out_hbm,
                 srci, dsti, srci1, dsti1, srci8, dsti8,
                 rows, rows1, rows8, zb, acc, sem, sem1):
    c = lax.axis_index("c")
    s = lax.axis_index("s")

    def zf(i, _):
        for kk in range(D // 16):
            zb[i, pl.ds(kk * 16, 16)] = jnp.zeros((16,), _f32)
        return 0
    lax.fori_loop(0, 64, zf, 0)
    for r in range(RPT // 64):
        pltpu.sync_copy(zb, acc.at[pl.ds(s * RPT + r * 64, 64)])
    plsc.subcore_barrier()

    eb = c * EPC + s * EPT

    def ld(j, sref, dref):
        pltpu.sync_copy(src_hbm.at[pl.ds(eb + j * CH, CH)], sref)
        pltpu.sync_copy(dst_hbm.at[pl.ds(eb + j * CH, CH)], dref)

    ld(0, srci, dsti)
    pltpu.async_copy(hn_hbm.at[srci], rows, sem)

    def body(k, _):
        j = 2 * k
        ld(j + 1, srci1, dsti1)
        pltpu.async_copy(hn_hbm.at[srci1], rows1, sem1)
        pltpu.make_async_copy(hn_hbm.at[srci], rows, sem).wait()
        pltpu.sync_copy(rows, acc.at[dsti], add=True)
        ld(j + 2, srci, dsti)
        pltpu.async_copy(hn_hbm.at[srci], rows, sem)
        pltpu.make_async_copy(hn_hbm.at[srci1], rows1, sem1).wait()
        pltpu.sync_copy(rows1, acc.at[dsti1], add=True)
        return 0
    lax.fori_loop(0, (NCH - 1) // 2, body, 0)
    pltpu.make_async_copy(hn_hbm.at[srci], rows, sem).wait()
    pltpu.sync_copy(rows, acc.at[dsti], add=True)

    pltpu.sync_copy(src_hbm.at[pl.ds(eb + NCH * CH, TL)], srci8)
    pltpu.sync_copy(dst_hbm.at[pl.ds(eb + NCH * CH, TL)], dsti8)
    pltpu.async_copy(hn_hbm.at[srci8], rows8, sem).wait()
    pltpu.sync_copy(rows8, acc.at[dsti8], add=True)

    plsc.subcore_barrier()
    pltpu.sync_copy(acc.at[pl.ds(s * RPT, RPT)],
                    out_hbm.at[pl.ds(c * NPAD + s * RPT, RPT)])


def _segsum_call():
    return _sc_kernel(
        "segsum",
        _segsum_body,
        out_type=jax.ShapeDtypeStruct((2 * NPAD, D), _f32),
        scratch_types=[
            pltpu.VMEM((CH,), jnp.int32),
            pltpu.VMEM((CH,), jnp.int32),
            pltpu.VMEM((CH,), jnp.int32),
            pltpu.VMEM((CH,), jnp.int32),
            pltpu.VMEM((TL,), jnp.int32),
            pltpu.VMEM((TL,), jnp.int32),
            pltpu.VMEM((CH, D), _f32),
            pltpu.VMEM((CH, D), _f32),
            pltpu.VMEM((TL, D), _f32),
            pltpu.VMEM((64, D), _f32),
            pltpu.VMEM_SHARED((NPAD, D), _f32),
            pltpu.SemaphoreType.DMA,
            pltpu.SemaphoreType.DMA,
        ],
    )


def _gath_body(f1_hbm, f2_hbm, src_hbm, dst_hbm,
               fs1_hbm, fs2_hbm, fd1_hbm, fd2_hbm,
               srci, dsti, srci1, dsti1, srci8, dsti8,
               s1b, s2b, d1b, d2b, s1b1, s2b1, d1b1, d2b1,
               s1b8, s2b8, d1b8, d2b8, sem, sem2, sem3, sem4):
    c = lax.axis_index("c")
    s = lax.axis_index("s")
    eb = (c * 16 + s) * EPT

    def ld(j, sref, dref):
        pltpu.sync_copy(src_hbm.at[pl.ds(eb + j * CH, CH)], sref)
        pltpu.sync_copy(dst_hbm.at[pl.ds(eb + j * CH, CH)], dref)

    def gat(sref, dref, b1, b2, b3, b4, sm, sm2):
        pltpu.async_copy(f1_hbm.at[sref], b1, sm)
        pltpu.async_copy(f2_hbm.at[sref], b2, sm)
        pltpu.async_copy(f1_hbm.at[dref], b3, sm2)
        pltpu.async_copy(f2_hbm.at[dref], b4, sm2)

    def wr(j, sref, dref, b1, b2, b3, b4, sm, sm2):
        pltpu.make_async_copy(f1_hbm.at[sref], b1, sm).wait()
        pltpu.make_async_copy(f2_hbm.at[sref], b2, sm).wait()
        pltpu.make_async_copy(f1_hbm.at[dref], b3, sm2).wait()
        pltpu.make_async_copy(f2_hbm.at[dref], b4, sm2).wait()
        pltpu.sync_copy(b1, fs1_hbm.at[pl.ds(eb + j * CH, CH)])
        pltpu.sync_copy(b2, fs2_hbm.at[pl.ds(eb + j * CH, CH)])
        pltpu.sync_copy(b3, fd1_hbm.at[pl.ds(eb + j * CH, CH)])
        pltpu.sync_copy(b4, fd2_hbm.at[pl.ds(eb + j * CH, CH)])

    ld(0, srci, dsti)
    gat(srci, dsti, s1b, s2b, d1b, d2b, sem, sem2)

    def body(k, _):
        j = 2 * k
        ld(j + 1, srci1, dsti1)
        gat(srci1, dsti1, s1b1, s2b1, d1b1, d2b1, sem3, sem4)
        wr(j, srci, dsti, s1b, s2b, d1b, d2b, sem, sem2)
        ld(j + 2, srci, dsti)
        gat(srci, dsti, s1b, s2b, d1b, d2b, sem, sem2)
        wr(j + 1, srci1, dsti1, s1b1, s2b1, d1b1, d2b1, sem3, sem4)
        return 0
    lax.fori_loop(0, (NCH - 1) // 2, body, 0)
    wr(NCH - 1, srci, dsti, s1b, s2b, d1b, d2b, sem, sem2)

    pltpu.sync_copy(src_hbm.at[pl.ds(eb + NCH * CH, TL)], srci8)
    pltpu.sync_copy(dst_hbm.at[pl.ds(eb + NCH * CH, TL)], dsti8)
    gat(srci8, dsti8, s1b8, s2b8, d1b8, d2b8, sem, sem2)
    pltpu.make_async_copy(f1_hbm.at[srci8], s1b8, sem).wait()
    pltpu.make_async_copy(f2_hbm.at[srci8], s2b8, sem).wait()
    pltpu.make_async_copy(f1_hbm.at[dsti8], d1b8, sem2).wait()
    pltpu.make_async_copy(f2_hbm.at[dsti8], d2b8, sem2).wait()
    pltpu.sync_copy(s1b8, fs1_hbm.at[pl.ds(eb + NCH * CH, TL)])
    pltpu.sync_copy(s2b8, fs2_hbm.at[pl.ds(eb + NCH * CH, TL)])
    pltpu.sync_copy(d1b8, fd1_hbm.at[pl.ds(eb + NCH * CH, TL)])
    pltpu.sync_copy(d2b8, fd2_hbm.at[pl.ds(eb + NCH * CH, TL)])


def _gath_call():
    return _sc_kernel(
        "gath",
        _gath_body,
        out_type=(jax.ShapeDtypeStruct((E, W1), _f32),
                  jax.ShapeDtypeStruct((E, W2), _f32),
                  jax.ShapeDtypeStruct((E, W1), _f32),
                  jax.ShapeDtypeStruct((E, W2), _f32)),
        scratch_types=[
            pltpu.VMEM((CH,), jnp.int32),
            pltpu.VMEM((CH,), jnp.int32),
            pltpu.VMEM((CH,), jnp.int32),
            pltpu.VMEM((CH,), jnp.int32),
            pltpu.VMEM((TL,), jnp.int32),
            pltpu.VMEM((TL,), jnp.int32),
            pltpu.VMEM((CH, W1), _f32),
            pltpu.VMEM((CH, W2), _f32),
            pltpu.VMEM((CH, W1), _f32),
            pltpu.VMEM((CH, W2), _f32),
            pltpu.VMEM((CH, W1), _f32),
            pltpu.VMEM((CH, W2), _f32),
            pltpu.VMEM((CH, W1), _f32),
            pltpu.VMEM((CH, W2), _f32),
            pltpu.VMEM((TL, W1), _f32),
            pltpu.VMEM((TL, W2), _f32),
            pltpu.VMEM((TL, W1), _f32),
            pltpu.VMEM((TL, W2), _f32),
            pltpu.SemaphoreType.DMA,
            pltpu.SemaphoreType.DMA,
            pltpu.SemaphoreType.DMA,
            pltpu.SemaphoreType.DMA,
        ],
    )


CHS = 40
NCHS = EPT // CHS


def _attscat_body(m1_hbm, m2_hbm, a_hbm, dst_hbm, z_hbm,
                  rs1_hbm, rs2_hbm, dn_hbm,
                  dsti, dsti1, m1b, m1b1, m2b, m2b1, ab, ab1, zb2, zba,
                  accm1, accm2, acca, sd0, sd1):
    c = lax.axis_index("c")
    s = lax.axis_index("s")

    pltpu.sync_copy(z_hbm.at[pl.ds(s * RPT, RPT)],
                    accm1.at[pl.ds(s * RPT, RPT)])

    def zf(i, _):
        zb2[i, pl.ds(0, 16)] = jnp.zeros((16,), _f32)
        zb2[i, pl.ds(16, 16)] = jnp.zeros((16,), _f32)
        zba[i, :] = jnp.zeros((16,), _f32)
        return 0
    lax.fori_loop(0, CHS, zf, 0)

    def zcp(r, _):
        pltpu.sync_copy(zb2, accm2.at[pl.ds(s * RPT + r * CHS, CHS)])
        pltpu.sync_copy(zba, acca.at[pl.ds(s * RPT + r * CHS, CHS)])
        return 0
    lax.fori_loop(0, RPT // CHS, zcp, 0)
    plsc.subcore_barrier()

    eb = c * EPC + s * EPT

    def lda(j, dref, b1, b2, ba, sm):
        rb = eb + j * CHS
        pltpu.async_copy(dst_hbm.at[pl.ds(rb, CHS)], dref, sm)
        pltpu.async_copy(m1_hbm.at[pl.ds(rb, CHS)], b1, sm)
        pltpu.async_copy(m2_hbm.at[pl.ds(rb, CHS)], b2, sm)
        pltpu.async_copy(a_hbm.at[pl.ds(rb, CHS)], ba, sm)

    def wsc(j, dref, b1, b2, ba, sm):
        rb = eb + j * CHS
        pltpu.make_async_copy(dst_hbm.at[pl.ds(rb, CHS)], dref, sm).wait()
        pltpu.make_async_copy(m1_hbm.at[pl.ds(rb, CHS)], b1, sm).wait()
        pltpu.make_async_copy(m2_hbm.at[pl.ds(rb, CHS)], b2, sm).wait()
        pltpu.make_async_copy(a_hbm.at[pl.ds(rb, CHS)], ba, sm).wait()
        pltpu.sync_copy(b1, accm1.at[dref], add=True)
        pltpu.sync_copy(b2, accm2.at[dref], add=True)
        pltpu.sync_copy(ba, acca.at[dref], add=True)

    lda(0, dsti, m1b, m2b, ab, sd0)

    def body(k, _):
        j = 2 * k
        lda(j + 1, dsti1, m1b1, m2b1, ab1, sd1)
        wsc(j, dsti, m1b, m2b, ab, sd0)
        lda(j + 2, dsti, m1b, m2b, ab, sd0)
        wsc(j + 1, dsti1, m1b1, m2b1, ab1, sd1)
        return 0
    lax.fori_loop(0, (NCHS - 1) // 2, body, 0)
    wsc(NCHS - 1, dsti, m1b, m2b, ab, sd0)

    plsc.subcore_barrier()
    pltpu.sync_copy(accm1.at[pl.ds(s * RPT, RPT)],
                    rs1_hbm.at[pl.ds(c * NPAD + s * RPT, RPT)])
    pltpu.sync_copy(accm2.at[pl.ds(s * RPT, RPT)],
                    rs2_hbm.at[pl.ds(c * NPAD + s * RPT, RPT)])
    pltpu.sync_copy(acca.at[pl.ds(s * RPT, RPT)],
                    dn_hbm.at[pl.ds(c * NPAD + s * RPT, RPT)])


def _attscat_call():
    return _sc_kernel(
        "attscat",
        _attscat_body,
        out_type=(jax.ShapeDtypeStruct((2 * NPAD, W1), _f32),
                  jax.ShapeDtypeStruct((2 * NPAD, W2), _f32),
                  jax.ShapeDtypeStruct((2 * NPAD, HP), _f32)),
        scratch_types=[
            pltpu.VMEM((CHS,), jnp.int32),
            pltpu.VMEM((CHS,), jnp.int32),
            pltpu.VMEM((CHS, W1), _f32),
            pltpu.VMEM((CHS, W1), _f32),
            pltpu.VMEM((CHS, W2), _f32),
            pltpu.VMEM((CHS, W2), _f32),
            pltpu.VMEM((CHS, HP), _f32),
            pltpu.VMEM((CHS, HP), _f32),
            pltpu.VMEM((CHS, W2), _f32),
            pltpu.VMEM((CHS, HP), _f32),
            pltpu.VMEM_SHARED((NPAD, W1), _f32),
            pltpu.VMEM_SHARED((NPAD, W2), _f32),
            pltpu.VMEM_SHARED((NPAD, HP), _f32),
            pltpu.SemaphoreType.DMA,
            pltpu.SemaphoreType.DMA,
        ],
    )


def _row_spec(w):
    return pl.BlockSpec((RB, w), lambda i: (i, 0))


def _full_spec(shape):
    nd = len(shape)
    return pl.BlockSpec(shape, lambda i: (0,) * nd)


def _dot(a, b):
    return jnp.dot(a, b, preferred_element_type=_f32)


def _in_body(x_ref, wi, bi, g0, b0, hn_ref):
    h = _dot(x_ref[...], wi[...]) + bi[...]
    hn_ref[...] = _lnorm(h, g0[...], b0[...])


_in_call = pl.pallas_call(
    _in_body,
    grid=(_GRID_N,),
    in_specs=[_row_spec(D), _full_spec((D, D)), _full_spec((1, D)),
              _full_spec((1, D)), _full_spec((1, D))],
    out_specs=_row_spec(D),
    out_shape=jax.ShapeDtypeStruct((NPAD, D), _f32),
)


def _make_conv_fin(has_feat):
    def body(hn_ref, sp_ref, degp_ref, wself, wneigh, bias, ig, ib, siw, sib,
             ng, nb, *rest):
        if has_feat:
            watt1, watt2 = rest[0], rest[1]
            outs = rest[2:]
        else:
            outs = rest
        hn = hn_ref[...]
        spv = sp_ref[...]
        sv = spv[0] + spv[1]
        dgv = degp_ref[...]
        deg = dgv[0, :, 0:1] + dgv[1, :, 0:1]
        mean = sv / jnp.maximum(deg, 1.0)
        conv = _dot(hn, wself[...]) + _dot(mean, wneigh[...]) + bias[...]
        h2 = _lnorm(conv + hn, ig[...], ib[...])
        hnew = h2 + _elu(_dot(h2, siw[...]) + sib[...])
        hn2 = _lnorm(hnew, ng[...], nb[...])
        outs[0][...] = hn2
        if has_feat:
            outs[1][...] = _dot(hn2, watt1[...])
            outs[2][...] = _dot(hn2, watt2[...])
    return body


def _conv_fin_call(has_feat):
    in_specs = [
        _row_spec(D),
        pl.BlockSpec((2, RB, D), lambda i: (0, i, 0)),
        pl.BlockSpec((2, RB, HP), lambda i: (0, i, 0)),
        _full_spec((D, D)), _full_spec((D, D)), _full_spec((1, D)),
        _full_spec((1, D)), _full_spec((1, D)),
        _full_spec((D, D)), _full_spec((1, D)),
        _full_spec((1, D)), _full_spec((1, D)),
    ]
    out_specs = [_row_spec(D)]
    out_shape = [jax.ShapeDtypeStruct((NPAD, D), _f32)]
    if has_feat:
        in_specs.append(_full_spec((D, W1)))
        in_specs.append(_full_spec((D, W2)))
        out_specs.append(_row_spec(W1))
        out_shape.append(jax.ShapeDtypeStruct((NPAD, W1), _f32))
        out_specs.append(_row_spec(W2))
        out_shape.append(jax.ShapeDtypeStruct((NPAD, W2), _f32))
    return pl.pallas_call(
        _make_conv_fin(has_feat),
        grid=(_GRID_N,),
        in_specs=in_specs,
        out_specs=out_specs,
        out_shape=out_shape,
    )


def _edge_body(fs1_ref, fs2_ref, fd1_ref, fd2_ref, ge1, ge2, gx1, gx2,
               m1_ref, m2_ref, a_ref):
    fs1 = fs1_ref[...]
    fs2 = fs2_ref[...]
    prod1 = fs1 * fd1_ref[...]
    prod2 = fs2 * fd2_ref[...]
    a = jnp.exp(_dot(prod1, ge1[...]) + _dot(prod2, ge2[...]))
    a_ref[...] = a
    m1_ref[...] = _dot(a, gx1[...]) * fs1
    m2_ref[...] = _dot(a, gx2[...]) * fs2


_edge_call = pl.pallas_call(
    _edge_body,
    grid=(_GRID_E,),
    in_specs=[pl.BlockSpec((EB, W1), lambda i: (i, 0)),
              pl.BlockSpec((EB, W2), lambda i: (i, 0)),
              pl.BlockSpec((EB, W1), lambda i: (i, 0)),
              pl.BlockSpec((EB, W2), lambda i: (i, 0)),
              _full_spec((W1, HP)), _full_spec((W2, HP)),
              _full_spec((HP, W1)), _full_spec((HP, W2))],
    out_specs=[pl.BlockSpec((EB, W1), lambda i: (i, 0)),
               pl.BlockSpec((EB, W2), lambda i: (i, 0)),
               pl.BlockSpec((EB, HP), lambda i: (i, 0))],
    out_shape=[jax.ShapeDtypeStruct((E, W1), _f32),
               jax.ShapeDtypeStruct((E, W2), _f32),
               jax.ShapeDtypeStruct((E, HP), _f32)],
)


def _make_att_fin(is_final):
    def body(hn_ref, rs1_ref, rs2_ref, dn_ref, gx1, gx2, hrw1, hrw2, hrb,
             ig, ib, f1w, f1b, f2w, f2b, *rest):
        hn = hn_ref[...]
        rs1v = rs1_ref[...]
        rs1 = rs1v[0] + rs1v[1]
        rs2v = rs2_ref[...]
        rs2 = rs2v[0] + rs2v[1]
        dnv = dn_ref[...]
        dn = dnv[0] + dnv[1]
        rst1 = rs1 / jnp.maximum(_dot(dn, gx1[...]), 1e-30)
        rst2 = rs2 / jnp.maximum(_dot(dn, gx2[...]), 1e-30)
        ho = (_dot(_elu(rst1), hrw1[...]) + _dot(_elu(rst2), hrw2[...])
              + hrb[...])
        h2 = _lnorm(ho + hn, ig[...], ib[...])
        ff = _elu(_dot(_elu(_dot(h2, f1w[...]) + f1b[...]), f2w[...])
                  + f2b[...])
        hnew = h2 + ff
        if is_final:
            wout, bout, out_ref = rest
            out_ref[...] = _dot(hnew, wout[...]) + bout[...]
        else:
            ng, nb, watt1, watt2, hn_out, f1_out, f2_out = rest
            hn2 = _lnorm(hnew, ng[...], nb[...])
            hn_out[...] = hn2
            f1_out[...] = _dot(hn2, watt1[...])
            f2_out[...] = _dot(hn2, watt2[...])
    return body


def _att_fin_call(is_final):
    in_specs = [
        _row_spec(D),
        pl.BlockSpec((2, RB, W1), lambda i: (0, i, 0)),
        pl.BlockSpec((2, RB, W2), lambda i: (0, i, 0)),
        pl.BlockSpec((2, RB, HP), lambda i: (0, i, 0)),
        _full_spec((HP, W1)), _full_spec((HP, W2)),
        _full_spec((W1, D)), _full_spec((W2, D)), _full_spec((1, D)),
        _full_spec((1, D)), _full_spec((1, D)),
        _full_spec((D, 4 * D)), _full_spec((1, 4 * D)),
        _full_spec((4 * D, D)), _full_spec((1, D)),
    ]
    if is_final:
        in_specs += [_full_spec((D, D)), _full_spec((1, D))]
        out_specs = _row_spec(D)
        out_shape = jax.ShapeDtypeStruct((NPAD, D), _f32)
    else:
        in_specs += [_full_spec((1, D)), _full_spec((1, D)),
                     _full_spec((D, W1)), _full_spec((D, W2))]
        out_specs = [_row_spec(D), _row_spec(W1), _row_spec(W2)]
        out_shape = [jax.ShapeDtypeStruct((NPAD, D), _f32),
                     jax.ShapeDtypeStruct((NPAD, W1), _f32),
                     jax.ShapeDtypeStruct((NPAD, W2), _f32)]
    return pl.pallas_call(
        _make_att_fin(is_final),
        grid=(_GRID_N,),
        in_specs=in_specs,
        out_specs=out_specs,
        out_shape=out_shape,
    )


_convfin_plain = _conv_fin_call(False)
_convfin_feat = _conv_fin_call(True)
_attfin_mid = _att_fin_call(False)
_attfin_last = _att_fin_call(True)


def _watt_pad(p):
    w = p["W_att"].reshape(D, H, DH)
    wp = jnp.pad(w, ((0, 0), (0, 0), (0, DHP - DH)))
    return (wp[:, :8].reshape(D, W1), wp[:, 8:].reshape(D, W2))


def _hrw_pad(p):
    w = p["hr_W"].reshape(H, DH, D)
    wp = jnp.pad(w, ((0, 0), (0, DHP - DH), (0, 0)))
    return (wp[:8].reshape(W1, D), wp[8:].reshape(W2, D))


def kernel(x, params, edge_index):
    src = edge_index[0]
    dst = edge_index[1]
    xp = jnp.pad(x, ((0, NPAD - N), (0, 0)))
    r1 = lambda v: v.reshape(1, -1)
    ge1 = jnp.asarray(_GE1_NP)
    ge2 = jnp.asarray(_GE2_NP)
    gx1 = jnp.asarray(_GX1_NP)
    gx2 = jnp.asarray(_GX2_NP)
    z1 = jnp.zeros((NPAD, W1), _f32)

    degp = _deg_call()(dst).reshape(2, NPAD, HP)

    p0 = params["conv0"]
    hn = _in_call(xp, params["W_in"], r1(params["b_in"]),
                  r1(p0["ln_g"]), r1(p0["ln_b"]))

    feat = None
    for i in range(3):
        p = params["conv%d" % i]
        sp = _segsum_call()(hn, src, dst).reshape(2, NPAD, D)
        common = (hn, sp, degp, p["Wself"], p["Wneigh"], r1(p["bias"]),
                  r1(p["iln_g"]), r1(p["iln_b"]), p["si_W"], r1(p["si_b"]))
        if i < 2:
            q = params["conv%d" % (i + 1)]
            (hn,) = _convfin_plain(*common, r1(q["ln_g"]), r1(q["ln_b"]))
        else:
            q = params["att0"]
            wa1, wa2 = _watt_pad(q)
            hn, feat1, feat2 = _convfin_feat(*common, r1(q["ln_g"]),
                                             r1(q["ln_b"]), wa1, wa2)

    for j in range(3):
        p = params["att%d" % j]
        fs1, fs2, fd1, fd2 = _gath_call()(feat1, feat2, src, dst)
        m1, m2, a = _edge_call(fs1, fs2, fd1, fd2, ge1, ge2, gx1, gx2)
        rs1, rs2, dnp = _attscat_call()(m1, m2, a, dst, z1)
        hr1, hr2 = _hrw_pad(p)
        common = (hn, rs1.reshape(2, NPAD, W1), rs2.reshape(2, NPAD, W2),
                  dnp.reshape(2, NPAD, HP),
                  gx1, gx2, hr1, hr2, r1(p["hr_b"]),
                  r1(p["iln_g"]), r1(p["iln_b"]),
                  p["ff1_W"], r1(p["ff1_b"]), p["ff2_W"], r1(p["ff2_b"]))
        if j < 2:
            q = params["att%d" % (j + 1)]
            wa1, wa2 = _watt_pad(q)
            hn, feat1, feat2 = _attfin_mid(*common, r1(q["ln_g"]),
                                           r1(q["ln_b"]), wa1, wa2)
        else:
            out = _attfin_last(*common, params["W_out"], r1(params["b_out"]))

    return out[:N]
